# baseline (speedup 1.0000x reference)
import functools

import jax
import jax.numpy as jnp
from jax import lax
from jax.experimental import pallas as pl
from jax.experimental.pallas import tpu as pltpu
from jax.experimental.pallas import tpu_sc as plsc

N = 10000
E = 320000
D = 128
H = 128
C = 4
NP = 10240

WS = 16

NC = 2
NS = 16
NW = NC * NS
EPW = E // NW
CK = 80
NCHUNK = EPW // CK
RPT = NP // NS

_f32 = jnp.float32

_mesh = plsc.VectorSubcoreMesh(core_axis_name="c", subcore_axis_name="s",
                               num_cores=NC, num_subcores=NS)


def _make_sc_agg(W, m, k, tiled):
  @functools.partial(
      pl.kernel,
      out_type=jax.ShapeDtypeStruct((NC, NP, W), _f32),
      mesh=_mesh,
      scratch_types=[
          pltpu.VMEM((EPW,), jnp.int32),
          [pltpu.VMEM((CK,), jnp.int32)] * m,
          [pltpu.VMEM((CK, W), _f32)] * m,
          [pltpu.SemaphoreType.DMA] * m,
          [pltpu.SemaphoreType.DMA] * m,
          [pltpu.SemaphoreType.DMA] * m,
          pltpu.VMEM_SHARED((NP, W), _f32),
      ],
      compiler_params=pltpu.CompilerParams(use_tc_tiling_on_sc=tiled),
  )
  def agg(z_hbm, src_hbm, dst_hbm, zero_hbm, out_hbm,
          sidx, didx, rows, gsems, dsems, ssems, acc):
    c = lax.axis_index("c")
    s = lax.axis_index("s")
    pltpu.sync_copy(zero_hbm, acc.at[pl.ds(s * RPT, RPT)])
    base = (s * NC + c) * EPW
    pltpu.sync_copy(src_hbm.at[pl.ds(base, EPW)], sidx)
    plsc.subcore_barrier()

    def issue_copies(i, b):
      pltpu.async_copy(dst_hbm.at[pl.ds(base + i * CK, CK)], didx[b], dsems[b])
      pltpu.async_copy(z_hbm.at[sidx.at[pl.ds(i * CK, CK)]], rows[b], gsems[b])

    def drain_scatter(b):
      pltpu.make_async_copy(rows[b], acc.at[didx[b]], ssems[b]).wait()

    def do_chunk(j, b, static_i=None):
      pltpu.make_async_copy(dst_hbm.at[pl.ds(base, CK)], didx[b],
                            dsems[b]).wait()
      pltpu.make_async_copy(z_hbm.at[sidx.at[pl.ds(j * CK, CK)]], rows[b],
                            gsems[b]).wait()
      pltpu.async_copy(rows[b], acc.at[didx[b]], ssems[b], add=True)
      bi = (b + k) % m
      if static_i is None:
        i = j + k

        @pl.when((i < NCHUNK) & (i >= m))
        def _():
          drain_scatter(bi)

        @pl.when(i < NCHUNK)
        def _():
          issue_copies(i, bi)
      elif static_i < NCHUNK:
        if static_i >= m:
          drain_scatter(bi)
        issue_copies(static_i, bi)

    for b in range(k):
      issue_copies(b, b)

    main_iters = NCHUNK // m

    def body(t, carry):
      for b in range(m):
        do_chunk(t * m + b, b)
      return carry

    lax.fori_loop(0, main_iters, body, 0)
    for j in range(main_iters * m, NCHUNK):
      do_chunk(j, j % m, static_i=j + k)
    for j in range(NCHUNK - m, NCHUNK):
      drain_scatter(j % m)
    plsc.subcore_barrier()
    pltpu.sync_copy(acc.at[pl.ds(s * RPT, RPT)],
                    out_hbm.at[c, pl.ds(s * RPT, RPT)])

  return agg


_M_CNT = 8
_K_CNT = 4


@functools.partial(
    pl.kernel,
    out_type=jax.ShapeDtypeStruct((NC, NP, WS), _f32),
    mesh=_mesh,
    scratch_types=[
        pltpu.VMEM((CK, WS), _f32),
        [pltpu.VMEM((CK,), jnp.int32)] * _M_CNT,
        [pltpu.SemaphoreType.DMA] * _M_CNT,
        [pltpu.SemaphoreType.DMA] * _M_CNT,
        pltpu.VMEM_SHARED((NP, WS), _f32),
    ],
    compiler_params=pltpu.CompilerParams(use_tc_tiling_on_sc=False),
)
def _sc_counts(dst_hbm, ones_hbm, zero_hbm, out_hbm,
               ones_v, didx, dsems, ssems, acc):
  c = lax.axis_index("c")
  s = lax.axis_index("s")
  pltpu.sync_copy(zero_hbm, acc.at[pl.ds(s * RPT, RPT)])
  pltpu.sync_copy(ones_hbm, ones_v)
  base = (s * NC + c) * EPW
  plsc.subcore_barrier()

  def issue_copies(i, b):
    pltpu.async_copy(dst_hbm.at[pl.ds(base + i * CK, CK)], didx[b], dsems[b])

  def drain_scatter(b):
    pltpu.make_async_copy(ones_v, acc.at[didx[b]], ssems[b]).wait()

  def do_chunk(b, static_i=None):
    pltpu.make_async_copy(dst_hbm.at[pl.ds(base, CK)], didx[b], dsems[b]).wait()
    pltpu.async_copy(ones_v, acc.at[didx[b]], ssems[b], add=True)
    bi = (b + _K_CNT) % _M_CNT
    if static_i is None:
      return bi
    if static_i < NCHUNK:
      if static_i >= _M_CNT:
        drain_scatter(bi)
      issue_copies(static_i, bi)

  for b in range(_K_CNT):
    issue_copies(b, b)

  main_iters = NCHUNK // _M_CNT

  def body(t, carry):
    for b in range(_M_CNT):
      j = t * _M_CNT + b
      bi = do_chunk(b)
      i = j + _K_CNT

      @pl.when((i < NCHUNK) & (i >= _M_CNT))
      def _():
        drain_scatter(bi)

      @pl.when(i < NCHUNK)
      def _():
        issue_copies(i, bi)

    return carry

  lax.fori_loop(0, main_iters, body, 0)
  for j in range(main_iters * _M_CNT, NCHUNK):
    do_chunk(j % _M_CNT, static_i=j + _K_CNT)
  for j in range(NCHUNK - _M_CNT, NCHUNK):
    drain_scatter(j % _M_CNT)
  plsc.subcore_barrier()
  pltpu.sync_copy(acc.at[pl.ds(s * RPT, RPT)],
                  out_hbm.at[c, pl.ds(s * RPT, RPT)])


_sc_agg_wide = _make_sc_agg(H, m=3, k=2, tiled=True)
_sc_agg_narrow = _make_sc_agg(WS, m=6, k=3, tiled=False)

_R = 2000
_G = N // _R


def _row_spec(w):
  return pl.BlockSpec((_R, w), lambda i: (i, 0))


def _pair_spec(w):
  return pl.BlockSpec((NC, _R, w), lambda i: (0, i, 0))


def _full_spec(shape):
  nd = len(shape)
  return pl.BlockSpec(shape, lambda i: (0,) * nd)


def _split_body(ei_ref, src_ref, dst_ref):
  src_ref[...] = ei_ref[0]
  dst_ref[...] = ei_ref[1]


def _tc_split_edges(ei):
  return pl.pallas_call(
      _split_body,
      out_shape=[jax.ShapeDtypeStruct((E,), jnp.int32),
                 jax.ShapeDtypeStruct((E,), jnp.int32)],
  )(ei)


def _dual_mm_body(x_ref, wl_ref, wr_ref, br_ref, zl_ref, zr_ref):
  xb = x_ref[...]
  zl_ref[...] = jnp.dot(xb, wl_ref[...], preferred_element_type=_f32)
  zr_ref[...] = jnp.dot(xb, wr_ref[...], preferred_element_type=_f32) + br_ref[...]


def _tc_dual_mm(x, wlT, wrT, br):
  wl_w, wr_w = wlT.shape[1], wrT.shape[1]
  return pl.pallas_call(
      _dual_mm_body,
      grid=(_G,),
      in_specs=[_row_spec(D), _full_spec(wlT.shape), _full_spec(wrT.shape),
                _full_spec(br.shape)],
      out_specs=[_row_spec(wl_w), _row_spec(wr_w)],
      out_shape=[jax.ShapeDtypeStruct((N, wl_w), _f32),
                 jax.ShapeDtypeStruct((N, wr_w), _f32)],
  )(x, wlT, wrT, br)


def _tc_combine1(p, q, zr, wlT, wrT, br):
  def body(p_ref, q_ref, zr_ref, wl_ref, wr_ref, br_ref,
           zl_ref, zro_ref, q_ref_out):
    qsum = q_ref[0] + q_ref[1]
    q_ref_out[...] = qsum
    cnt = jnp.maximum(qsum[:, 0:1], 1.0)
    h = (p_ref[0] + p_ref[1]) / cnt + zr_ref[...]
    h = jnp.maximum(h, 0.0)
    zl_ref[...] = jnp.dot(h, wl_ref[...], preferred_element_type=_f32)
    zro_ref[...] = jnp.dot(h, wr_ref[...], preferred_element_type=_f32) + br_ref[...]

  return pl.pallas_call(
      body,
      grid=(_G,),
      in_specs=[_pair_spec(H), _pair_spec(WS), _row_spec(H),
                _full_spec(wlT.shape), _full_spec(wrT.shape),
                _full_spec(br.shape)],
      out_specs=[_row_spec(H), _row_spec(H), _row_spec(WS)],
      out_shape=[jax.ShapeDtypeStruct((N, H), _f32),
                 jax.ShapeDtypeStruct((N, H), _f32),
                 jax.ShapeDtypeStruct((N, WS), _f32)],
  )(p, q, zr, wlT, wrT, br)


def _tc_combine2(p, cnt16, zr, wlT, wrT, br):
  def body(p_ref, q_ref, zr_ref, wl_ref, wr_ref, br_ref,
           h_ref, zl_ref, zro_ref):
    cnt = jnp.maximum(q_ref[:, 0:1], 1.0)
    h = (p_ref[0] + p_ref[1]) / cnt + zr_ref[...]
    h_ref[...] = h
    zl_ref[...] = jnp.dot(h, wl_ref[...], preferred_element_type=_f32)
    zro_ref[...] = jnp.dot(h, wr_ref[...], preferred_element_type=_f32) + br_ref[...]

  return pl.pallas_call(
      body,
      grid=(_G,),
      in_specs=[_pair_spec(H), _row_spec(WS), _row_spec(H),
                _full_spec(wlT.shape), _full_spec(wrT.shape),
                _full_spec(br.shape)],
      out_specs=[_row_spec(H), _row_spec(WS), _row_spec(WS)],
      out_shape=[jax.ShapeDtypeStruct((N, H), _f32),
                 jax.ShapeDtypeStruct((N, WS), _f32),
                 jax.ShapeDtypeStruct((N, WS), _f32)],
  )(p, cnt16, zr, wlT, wrT, br)


def _final_body(p_ref, q_ref, zr_ref, out_ref):
  cnt = jnp.maximum(q_ref[:, 0:1], 1.0)
  out_ref[...] = (p_ref[0] + p_ref[1]) / cnt + zr_ref[...]


def _tc_final(p, cnt16, zr):
  return pl.pallas_call(
      _final_body,
      grid=(_G,),
      in_specs=[_pair_spec(WS), _row_spec(WS), _row_spec(WS)],
      out_specs=_row_spec(WS),
      out_shape=jax.ShapeDtypeStruct((N, WS), _f32),
  )(p, cnt16, zr)


def kernel(x, W0l, b0, W0r, W1l, b1, W1r, W2l, b2, W2r, edge_index):
  def padT(w, width):
    wT = w.T.astype(_f32)
    return jnp.pad(wT, ((0, 0), (0, width - wT.shape[1])))

  wl0T = W0l.T.astype(_f32)
  wr0T = W0r.T.astype(_f32)
  wl1T = W1l.T.astype(_f32)
  wr1T = W1r.T.astype(_f32)
  wl2T = padT(W2l, WS)
  wr2T = padT(W2r, WS)
  br2 = jnp.pad(b2.astype(_f32), (0, WS - C)).reshape(1, WS)

  zeros_wide = jnp.zeros((RPT, H), _f32)
  zeros_narrow = jnp.zeros((RPT, WS), _f32)
  ones_rows = jnp.ones((CK, WS), _f32)

  src, dst = _tc_split_edges(edge_index)
  q = _sc_counts(dst, ones_rows, zeros_narrow)
  zl0, zr0 = _tc_dual_mm(x, wl0T, wr0T, b0.reshape(1, H))
  p0 = _sc_agg_wide(zl0, src, dst, zeros_wide)
  zl1, zr1, cnt16 = _tc_combine1(p0, q, zr0, wl1T, wr1T, b1.reshape(1, H))
  p1 = _sc_agg_wide(zl1, src, dst, zeros_wide)
  h1, zl2, zr2 = _tc_combine2(p1, cnt16, zr1, wl2T, wr2T, br2)
  p2 = _sc_agg_narrow(zl2, src, dst, zeros_narrow)
  out = _tc_final(p2, cnt16, zr2)[:, :C]
  return (out, out, h1)


# speedup vs baseline: 1.0571x; 1.0571x over previous
import functools

import jax
import jax.numpy as jnp
from jax import lax
from jax.experimental import pallas as pl
from jax.experimental.pallas import tpu as pltpu
from jax.experimental.pallas import tpu_sc as plsc

N = 10000
E = 320000
D = 128
H = 128
C = 4
NP = 10240

WS = 16

NC = 2
NS = 16
NW = NC * NS
EPW = E // NW
CK = 80
NCHUNK = EPW // CK
RPT = NP // NS

_f32 = jnp.float32

_mesh = plsc.VectorSubcoreMesh(core_axis_name="c", subcore_axis_name="s",
                               num_cores=NC, num_subcores=NS)


def _make_sc_agg(W, m, tiled):
  @functools.partial(
      pl.kernel,
      out_type=jax.ShapeDtypeStruct((NC, NP, W), _f32),
      mesh=_mesh,
      scratch_types=[
          pltpu.VMEM((EPW,), jnp.int32),
          [pltpu.VMEM((CK,), jnp.int32)] * m,
          [pltpu.VMEM((CK, W), _f32)] * m,
          [pltpu.SemaphoreType.DMA] * m,
          [pltpu.SemaphoreType.DMA] * m,
          pltpu.VMEM_SHARED((NP, W), _f32),
      ],
      compiler_params=pltpu.CompilerParams(use_tc_tiling_on_sc=tiled),
  )
  def agg(z_hbm, src_hbm, dst_hbm, zero_hbm, out_hbm,
          sidx, didx, rows, gsems, dsems, acc):
    c = lax.axis_index("c")
    s = lax.axis_index("s")
    pltpu.sync_copy(zero_hbm, acc.at[pl.ds(s * RPT, RPT)])
    base = (s * NC + c) * EPW
    pltpu.sync_copy(src_hbm.at[pl.ds(base, EPW)], sidx)
    plsc.subcore_barrier()

    def start(j, b):
      pltpu.async_copy(dst_hbm.at[pl.ds(base + j * CK, CK)], didx[b], dsems[b])
      pltpu.async_copy(z_hbm.at[sidx.at[pl.ds(j * CK, CK)]], rows[b], gsems[b])

    def finish(j, b):
      pltpu.make_async_copy(dst_hbm.at[pl.ds(base, CK)], didx[b],
                            dsems[b]).wait()
      pltpu.make_async_copy(z_hbm.at[sidx.at[pl.ds(j * CK, CK)]], rows[b],
                            gsems[b]).wait()
      pltpu.sync_copy(rows[b], acc.at[didx[b]], add=True)

    for b in range(m):
      start(b, b)

    main_iters = NCHUNK // m

    def body(t, carry):
      for b in range(m):
        j = t * m + b
        finish(j, b)

        @pl.when(j + m < NCHUNK)
        def _():
          start(j + m, b)

      return carry

    lax.fori_loop(0, main_iters, body, 0)
    for j in range(main_iters * m, NCHUNK):
      finish(j, j % m)
    plsc.subcore_barrier()
    pltpu.sync_copy(acc.at[pl.ds(s * RPT, RPT)],
                    out_hbm.at[c, pl.ds(s * RPT, RPT)])

  return agg


_M_CNT = 8
_K_CNT = 4


@functools.partial(
    pl.kernel,
    out_type=jax.ShapeDtypeStruct((NC, NP, WS), _f32),
    mesh=_mesh,
    scratch_types=[
        pltpu.VMEM((CK, WS), _f32),
        [pltpu.VMEM((CK,), jnp.int32)] * _M_CNT,
        [pltpu.SemaphoreType.DMA] * _M_CNT,
        pltpu.VMEM_SHARED((NP, WS), _f32),
    ],
    compiler_params=pltpu.CompilerParams(use_tc_tiling_on_sc=False),
)
def _sc_counts(dst_hbm, ones_hbm, zero_hbm, out_hbm,
               ones_v, didx, dsems, acc):
  c = lax.axis_index("c")
  s = lax.axis_index("s")
  pltpu.sync_copy(zero_hbm, acc.at[pl.ds(s * RPT, RPT)])
  pltpu.sync_copy(ones_hbm, ones_v)
  base = (s * NC + c) * EPW
  plsc.subcore_barrier()

  def start(j, b):
    pltpu.async_copy(dst_hbm.at[pl.ds(base + j * CK, CK)], didx[b], dsems[b])

  def finish(b):
    pltpu.make_async_copy(dst_hbm.at[pl.ds(base, CK)], didx[b], dsems[b]).wait()
    pltpu.sync_copy(ones_v, acc.at[didx[b]], add=True)

  for b in range(_M_CNT):
    start(b, b)

  main_iters = NCHUNK // _M_CNT

  def body(t, carry):
    for b in range(_M_CNT):
      j = t * _M_CNT + b
      finish(b)

      @pl.when(j + _M_CNT < NCHUNK)
      def _():
        start(j + _M_CNT, b)

    return carry

  lax.fori_loop(0, main_iters, body, 0)
  for j in range(main_iters * _M_CNT, NCHUNK):
    finish(j % _M_CNT)
  plsc.subcore_barrier()
  pltpu.sync_copy(acc.at[pl.ds(s * RPT, RPT)],
                  out_hbm.at[c, pl.ds(s * RPT, RPT)])


_sc_agg_wide = _make_sc_agg(H, m=3, tiled=True)
_sc_agg_narrow = _make_sc_agg(WS, m=6, tiled=False)

_R = 2000
_G = N // _R


def _row_spec(w):
  return pl.BlockSpec((_R, w), lambda i: (i, 0))


def _pair_spec(w):
  return pl.BlockSpec((NC, _R, w), lambda i: (0, i, 0))


def _full_spec(shape):
  nd = len(shape)
  return pl.BlockSpec(shape, lambda i: (0,) * nd)


def _split_body(ei_ref, src_ref, dst_ref):
  src_ref[...] = ei_ref[0]
  dst_ref[...] = ei_ref[1]


def _tc_split_edges(ei):
  return pl.pallas_call(
      _split_body,
      out_shape=[jax.ShapeDtypeStruct((E,), jnp.int32),
                 jax.ShapeDtypeStruct((E,), jnp.int32)],
  )(ei)


def _dual_mm_body(x_ref, wl_ref, wr_ref, br_ref, zl_ref, zr_ref):
  xb = x_ref[...]
  zl_ref[...] = jnp.dot(xb, wl_ref[...], preferred_element_type=_f32)
  zr_ref[...] = jnp.dot(xb, wr_ref[...], preferred_element_type=_f32) + br_ref[...]


def _tc_dual_mm(x, wlT, wrT, br):
  wl_w, wr_w = wlT.shape[1], wrT.shape[1]
  return pl.pallas_call(
      _dual_mm_body,
      grid=(_G,),
      in_specs=[_row_spec(D), _full_spec(wlT.shape), _full_spec(wrT.shape),
                _full_spec(br.shape)],
      out_specs=[_row_spec(wl_w), _row_spec(wr_w)],
      out_shape=[jax.ShapeDtypeStruct((N, wl_w), _f32),
                 jax.ShapeDtypeStruct((N, wr_w), _f32)],
  )(x, wlT, wrT, br)


def _tc_combine1(p, q, zr, wlT, wrT, br):
  def body(p_ref, q_ref, zr_ref, wl_ref, wr_ref, br_ref,
           zl_ref, zro_ref, q_ref_out):
    qsum = q_ref[0] + q_ref[1]
    q_ref_out[...] = qsum
    cnt = jnp.maximum(qsum[:, 0:1], 1.0)
    h = (p_ref[0] + p_ref[1]) / cnt + zr_ref[...]
    h = jnp.maximum(h, 0.0)
    zl_ref[...] = jnp.dot(h, wl_ref[...], preferred_element_type=_f32)
    zro_ref[...] = jnp.dot(h, wr_ref[...], preferred_element_type=_f32) + br_ref[...]

  return pl.pallas_call(
      body,
      grid=(_G,),
      in_specs=[_pair_spec(H), _pair_spec(WS), _row_spec(H),
                _full_spec(wlT.shape), _full_spec(wrT.shape),
                _full_spec(br.shape)],
      out_specs=[_row_spec(H), _row_spec(H), _row_spec(WS)],
      out_shape=[jax.ShapeDtypeStruct((N, H), _f32),
                 jax.ShapeDtypeStruct((N, H), _f32),
                 jax.ShapeDtypeStruct((N, WS), _f32)],
  )(p, q, zr, wlT, wrT, br)


def _tc_combine2(p, cnt16, zr, wlT, wrT, br):
  def body(p_ref, q_ref, zr_ref, wl_ref, wr_ref, br_ref,
           h_ref, zl_ref, zro_ref):
    cnt = jnp.maximum(q_ref[:, 0:1], 1.0)
    h = (p_ref[0] + p_ref[1]) / cnt + zr_ref[...]
    h_ref[...] = h
    zl_ref[...] = jnp.dot(h, wl_ref[...], preferred_element_type=_f32)
    zro_ref[...] = jnp.dot(h, wr_ref[...], preferred_element_type=_f32) + br_ref[...]

  return pl.pallas_call(
      body,
      grid=(_G,),
      in_specs=[_pair_spec(H), _row_spec(WS), _row_spec(H),
                _full_spec(wlT.shape), _full_spec(wrT.shape),
                _full_spec(br.shape)],
      out_specs=[_row_spec(H), _row_spec(WS), _row_spec(WS)],
      out_shape=[jax.ShapeDtypeStruct((N, H), _f32),
                 jax.ShapeDtypeStruct((N, WS), _f32),
                 jax.ShapeDtypeStruct((N, WS), _f32)],
  )(p, cnt16, zr, wlT, wrT, br)


def _final_body(p_ref, q_ref, zr_ref, out_ref):
  cnt = jnp.maximum(q_ref[:, 0:1], 1.0)
  out_ref[...] = (p_ref[0] + p_ref[1]) / cnt + zr_ref[...]


def _tc_final(p, cnt16, zr):
  return pl.pallas_call(
      _final_body,
      grid=(_G,),
      in_specs=[_pair_spec(WS), _row_spec(WS), _row_spec(WS)],
      out_specs=_row_spec(WS),
      out_shape=jax.ShapeDtypeStruct((N, WS), _f32),
  )(p, cnt16, zr)


def kernel(x, W0l, b0, W0r, W1l, b1, W1r, W2l, b2, W2r, edge_index):
  def padT(w, width):
    wT = w.T.astype(_f32)
    return jnp.pad(wT, ((0, 0), (0, width - wT.shape[1])))

  wl0T = W0l.T.astype(_f32)
  wr0T = W0r.T.astype(_f32)
  wl1T = W1l.T.astype(_f32)
  wr1T = W1r.T.astype(_f32)
  wl2T = padT(W2l, WS)
  wr2T = padT(W2r, WS)
  br2 = jnp.pad(b2.astype(_f32), (0, WS - C)).reshape(1, WS)

  zeros_wide = jnp.zeros((RPT, H), _f32)
  zeros_narrow = jnp.zeros((RPT, WS), _f32)
  ones_rows = jnp.ones((CK, WS), _f32)

  src, dst = _tc_split_edges(edge_index)
  q = _sc_counts(dst, ones_rows, zeros_narrow)
  zl0, zr0 = _tc_dual_mm(x, wl0T, wr0T, b0.reshape(1, H))
  p0 = _sc_agg_wide(zl0, src, dst, zeros_wide)
  zl1, zr1, cnt16 = _tc_combine1(p0, q, zr0, wl1T, wr1T, b1.reshape(1, H))
  p1 = _sc_agg_wide(zl1, src, dst, zeros_wide)
  h1, zl2, zr2 = _tc_combine2(p1, cnt16, zr1, wl2T, wr2T, br2)
  p2 = _sc_agg_narrow(zl2, src, dst, zeros_narrow)
  out = _tc_final(p2, cnt16, zr2)[:, :C]
  return (out, out, h1)
